# Initial kernel scaffold; baseline (speedup 1.0000x reference)
import functools

import jax
import jax.numpy as jnp
from jax import lax
from jax.experimental import pallas as pl
from jax.experimental.pallas import tpu as pltpu
from jax.experimental.pallas import tpu_sc as plsc

N = 10000
E = 320000
D = 128

NC = 2
NS = 16
NW = NC * NS
L = 16

CH = 128
G = -(-E // (NW * CH))
E_PAD = NW * G * CH
N_PAD = 10240
RPT = N_PAD // NS

_MESH = dict(core_axis_name="c", subcore_axis_name="s", num_cores=NC,
             num_subcores=NS)

_ZERO16 = jnp.zeros((L,), jnp.float32)


def _deg_body(src_hbm, ew_hbm, degp_hbm, src_v, ew_v, z_v, deg_sh):
    c = lax.axis_index("c")
    s = lax.axis_index("s")
    wid = s * NC + c

    def zero_z(i, carry):
        z_v[pl.ds(i * L, L)] = _ZERO16
        return carry

    lax.fori_loop(0, RPT // L, zero_z, 0)
    pltpu.sync_copy(z_v, deg_sh.at[pl.ds(s * RPT, RPT)])
    plsc.subcore_barrier()

    def chunk(g, carry):
        base = (wid * G + g) * CH
        pltpu.sync_copy(src_hbm.at[pl.ds(base, CH)], src_v)
        pltpu.sync_copy(ew_hbm.at[pl.ds(base, CH)], ew_v)
        pltpu.sync_copy(ew_v, deg_sh.at[src_v], add=True)
        return carry

    lax.fori_loop(0, G, chunk, 0)
    plsc.subcore_barrier()
    pltpu.sync_copy(deg_sh.at[pl.ds(s * RPT, RPT)],
                    degp_hbm.at[c, pl.ds(s * RPT, RPT)])


_deg_call = pl.kernel(
    _deg_body,
    out_type=jax.ShapeDtypeStruct((NC, N_PAD), jnp.float32),
    mesh=plsc.VectorSubcoreMesh(**_MESH),
    scratch_types=[
        pltpu.VMEM((CH,), jnp.int32),
        pltpu.VMEM((CH,), jnp.float32),
        pltpu.VMEM((RPT,), jnp.float32),
        pltpu.VMEM_SHARED((N_PAD,), jnp.float32),
    ],
)


def _dis_body(degp_ref, dis_ref):
    d = degp_ref[0] + degp_ref[1]
    dis_ref[...] = jnp.where(d > 0, lax.rsqrt(d), 0.0)


_dis_call = pl.pallas_call(
    _dis_body,
    out_shape=jax.ShapeDtypeStruct((N_PAD // D, D), jnp.float32),
)


def _prop_body(src_hbm, dst_hbm, ew_hbm, dis_hbm, x_hbm, pp_hbm,
               src_v, dst_v, ew_v, w_v, dis_v, rows_v, acc_sh, sem):
    c = lax.axis_index("c")
    s = lax.axis_index("s")
    wid = s * NC + c

    pltpu.sync_copy(dis_hbm, dis_v)

    def zero_rows(r, carry):
        for cs in range(D // L):
            rows_v[r, pl.ds(cs * L, L)] = _ZERO16
        return carry

    lax.fori_loop(0, CH, zero_rows, 0)
    for t in range(RPT // CH):
        pltpu.sync_copy(rows_v, acc_sh.at[pl.ds(s * RPT + t * CH, CH)])
    plsc.subcore_barrier()

    def chunk(g, carry):
        base = (wid * G + g) * CH
        pltpu.sync_copy(src_hbm.at[pl.ds(base, CH)], src_v)
        pltpu.sync_copy(dst_hbm.at[pl.ds(base, CH)], dst_v)
        pltpu.sync_copy(ew_hbm.at[pl.ds(base, CH)], ew_v)
        pltpu.async_copy(x_hbm.at[src_v], rows_v, sem).wait()
        for t in range(CH // L):
            s16 = src_v[pl.ds(t * L, L)]
            d16 = dst_v[pl.ds(t * L, L)]
            e16 = ew_v[pl.ds(t * L, L)]
            w16 = -(plsc.load_gather(dis_v, [s16]) * e16
                    * plsc.load_gather(dis_v, [d16]))
            w_v[...] = w16
            for j in range(L):
                wj = plsc.load_gather(w_v, [jnp.full((L,), j, jnp.int32)])
                e = t * L + j
                for cs in range(D // L):
                    rows_v[e, pl.ds(cs * L, L)] = (
                        rows_v[e, pl.ds(cs * L, L)] * wj)
        pltpu.sync_copy(rows_v, acc_sh.at[dst_v], add=True)
        return carry

    lax.fori_loop(0, G, chunk, 0)
    plsc.subcore_barrier()
    for t in range(RPT // CH):
        pltpu.sync_copy(acc_sh.at[pl.ds(s * RPT + t * CH, CH)],
                        pp_hbm.at[c, pl.ds(s * RPT + t * CH, CH)])


_prop_call = pl.kernel(
    _prop_body,
    out_type=jax.ShapeDtypeStruct((NC, N_PAD, D), jnp.float32),
    mesh=plsc.VectorSubcoreMesh(**_MESH),
    scratch_types=[
        pltpu.VMEM((CH,), jnp.int32),
        pltpu.VMEM((CH,), jnp.int32),
        pltpu.VMEM((CH,), jnp.float32),
        pltpu.VMEM((L,), jnp.float32),
        pltpu.VMEM((N_PAD,), jnp.float32),
        pltpu.VMEM((CH, D), jnp.float32),
        pltpu.VMEM_SHARED((N_PAD, D), jnp.float32),
        pltpu.SemaphoreType.DMA,
    ],
)


BR = 1024


def _gates_body(pp_ref, x_ref, w_ref, v_ref, bxs_ref, addb_ref, h_ref, c_ref):
    P = pp_ref[0] + pp_ref[1]
    x = x_ref[...]

    def gate(k):
        y = jnp.dot(P, w_ref[k], preferred_element_type=jnp.float32)
        y = y + jnp.dot(x, v_ref[k], preferred_element_type=jnp.float32)
        return jnp.maximum(y + bxs_ref[k][None, :], 0.0) + addb_ref[k][None, :]

    i_g = jax.nn.sigmoid(gate(0))
    t_g = jnp.tanh(gate(1))
    o_g = jax.nn.sigmoid(gate(2))
    cc = i_g * t_g
    h_ref[...] = o_g * jnp.tanh(cc)
    c_ref[...] = cc


_gates_call = pl.pallas_call(
    _gates_body,
    grid=(N_PAD // BR,),
    in_specs=[
        pl.BlockSpec((NC, BR, D), lambda i: (0, i, 0)),
        pl.BlockSpec((BR, D), lambda i: (i, 0)),
        pl.BlockSpec((3, D, D), lambda i: (0, 0, 0)),
        pl.BlockSpec((3, D, D), lambda i: (0, 0, 0)),
        pl.BlockSpec((8, D), lambda i: (0, 0)),
        pl.BlockSpec((8, D), lambda i: (0, 0)),
    ],
    out_specs=[
        pl.BlockSpec((BR, D), lambda i: (i, 0)),
        pl.BlockSpec((BR, D), lambda i: (i, 0)),
    ],
    out_shape=[
        jax.ShapeDtypeStruct((N_PAD, D), jnp.float32),
        jax.ShapeDtypeStruct((N_PAD, D), jnp.float32),
    ],
)


def kernel(X, edge_index, edge_weight, Wx, Vx, bx, Wh, Vh, bh, b):
    src = edge_index[0]
    dst = edge_index[1]

    npad = E_PAD - E
    pad_ids = jnp.arange(npad, dtype=jnp.int32)
    src_p = jnp.concatenate([src, pad_ids % N])
    dst_p = jnp.concatenate([dst, N + pad_ids % (N_PAD - N)])
    ew_p = jnp.concatenate([edge_weight,
                            jnp.zeros((npad,), jnp.float32)])

    degp = _deg_call(src_p, ew_p)
    dis = _dis_call(degp.reshape(NC, N_PAD // D, D)).reshape(N_PAD)
    pp = _prop_call(src_p, dst_p, ew_p, dis, X)

    Xp = jnp.pad(X, ((0, N_PAD - N), (0, 0)))
    sel = jnp.array([0, 2, 3])
    bxs = jnp.pad(bx[sel], ((0, 5), (0, 0)))
    addb = jnp.pad(jax.nn.relu(bh[sel]) + b[sel, 0], ((0, 5), (0, 0)))
    H, C = _gates_call(pp, Xp, Wx[sel], Vx[sel], bxs, addb)
    return H[:N], C[:N]


# speedup vs baseline: 35.9788x; 35.9788x over previous
import functools

import jax
import jax.numpy as jnp
from jax import lax
from jax.experimental import pallas as pl
from jax.experimental.pallas import tpu as pltpu
from jax.experimental.pallas import tpu_sc as plsc

N = 10000
E = 320000
D = 128

NC = 2
NS = 16
NW = NC * NS
L = 16

CH = 128
G = -(-E // (NW * CH))
E_PAD = NW * G * CH
N_PAD = 10240
RPT = N_PAD // NS

_MESH = dict(core_axis_name="c", subcore_axis_name="s", num_cores=NC,
             num_subcores=NS)


def _deg_body(src_hbm, ew_hbm, degp_hbm, src_v, ew_v, z_v, deg_sh):
    c = lax.axis_index("c")
    s = lax.axis_index("s")
    wid = s * NC + c

    zero16 = jnp.zeros((L,), jnp.float32)

    def zero_z(i, carry):
        z_v[pl.ds(i * L, L)] = zero16
        return carry

    lax.fori_loop(0, RPT // L, zero_z, 0)
    pltpu.sync_copy(z_v, deg_sh.at[pl.ds(s * RPT, RPT)])
    plsc.subcore_barrier()

    def chunk(g, carry):
        base = (wid * G + g) * CH
        pltpu.sync_copy(src_hbm.at[pl.ds(base, CH)], src_v)
        pltpu.sync_copy(ew_hbm.at[pl.ds(base, CH)], ew_v)
        pltpu.sync_copy(ew_v, deg_sh.at[src_v], add=True)
        return carry

    lax.fori_loop(0, G, chunk, 0)
    plsc.subcore_barrier()
    pltpu.sync_copy(deg_sh.at[pl.ds(s * RPT, RPT)],
                    degp_hbm.at[c, pl.ds(s * RPT, RPT)])


_deg_call = pl.kernel(
    _deg_body,
    out_type=jax.ShapeDtypeStruct((NC, N_PAD), jnp.float32),
    mesh=plsc.VectorSubcoreMesh(**_MESH),
    scratch_types=[
        pltpu.VMEM((CH,), jnp.int32),
        pltpu.VMEM((CH,), jnp.float32),
        pltpu.VMEM((RPT,), jnp.float32),
        pltpu.VMEM_SHARED((N_PAD,), jnp.float32),
    ],
)


def _dis_body(degp_ref, dis_ref):
    d = degp_ref[0] + degp_ref[1]
    dis_ref[...] = jnp.where(d > 0, lax.rsqrt(d), 0.0)


_dis_call = pl.pallas_call(
    _dis_body,
    out_shape=jax.ShapeDtypeStruct((N_PAD // D, D), jnp.float32),
)


def _prop_body(src_hbm, dst_hbm, ew_hbm, dis_hbm, x_hbm, pp_hbm,
               src_v, dst_v, ew_v, dis_v, rows_v, acc_sh, sem):
    c = lax.axis_index("c")
    s = lax.axis_index("s")
    wid = s * NC + c

    pltpu.sync_copy(dis_hbm, dis_v)

    zero16 = jnp.zeros((L,), jnp.float32)

    def zero_rows(r, carry):
        for cs in range(D // L):
            rows_v[r, pl.ds(cs * L, L)] = zero16
        return carry

    lax.fori_loop(0, CH, zero_rows, 0)
    for t in range(RPT // CH):
        pltpu.sync_copy(rows_v, acc_sh.at[pl.ds(s * RPT + t * CH, CH)])
    plsc.subcore_barrier()

    def chunk(g, carry):
        base = (wid * G + g) * CH
        pltpu.sync_copy(src_hbm.at[pl.ds(base, CH)], src_v)
        pltpu.sync_copy(dst_hbm.at[pl.ds(base, CH)], dst_v)
        pltpu.sync_copy(ew_hbm.at[pl.ds(base, CH)], ew_v)
        pltpu.async_copy(x_hbm.at[src_v], rows_v, sem).wait()
        for t in range(CH // L):
            s16 = src_v[pl.ds(t * L, L)]
            d16 = dst_v[pl.ds(t * L, L)]
            e16 = ew_v[pl.ds(t * L, L)]
            w16 = -(plsc.load_gather(dis_v, [s16]) * e16
                    * plsc.load_gather(dis_v, [d16]))
            for j in range(L):
                wj = lax.gather(
                    w16, jnp.full((L, 1), j, jnp.int32),
                    lax.GatherDimensionNumbers(
                        offset_dims=(), collapsed_slice_dims=(0,),
                        start_index_map=(0,)),
                    (1,), mode=lax.GatherScatterMode.PROMISE_IN_BOUNDS)
                e = t * L + j
                for cs in range(D // L):
                    rows_v[e, pl.ds(cs * L, L)] = (
                        rows_v[e, pl.ds(cs * L, L)] * wj)
        pltpu.sync_copy(rows_v, acc_sh.at[dst_v], add=True)
        return carry

    lax.fori_loop(0, G, chunk, 0)
    plsc.subcore_barrier()
    for t in range(RPT // CH):
        pltpu.sync_copy(acc_sh.at[pl.ds(s * RPT + t * CH, CH)],
                        pp_hbm.at[c, pl.ds(s * RPT + t * CH, CH)])


_prop_call = pl.kernel(
    _prop_body,
    out_type=jax.ShapeDtypeStruct((NC, N_PAD, D), jnp.float32),
    mesh=plsc.VectorSubcoreMesh(**_MESH),
    scratch_types=[
        pltpu.VMEM((CH,), jnp.int32),
        pltpu.VMEM((CH,), jnp.int32),
        pltpu.VMEM((CH,), jnp.float32),
        pltpu.VMEM((N_PAD,), jnp.float32),
        pltpu.VMEM((CH, D), jnp.float32),
        pltpu.VMEM_SHARED((N_PAD, D), jnp.float32),
        pltpu.SemaphoreType.DMA,
    ],
    compiler_params=pltpu.CompilerParams(needs_layout_passes=False),
)


BR = 1024


def _gates_body(pp_ref, x_ref, w_ref, v_ref, bxs_ref, addb_ref, h_ref, c_ref):
    P = pp_ref[0] + pp_ref[1]
    x = x_ref[...]

    def gate(k):
        y = jnp.dot(P, w_ref[k], preferred_element_type=jnp.float32)
        y = y + jnp.dot(x, v_ref[k], preferred_element_type=jnp.float32)
        return jnp.maximum(y + bxs_ref[k][None, :], 0.0) + addb_ref[k][None, :]

    i_g = jax.nn.sigmoid(gate(0))
    t_g = jnp.tanh(gate(1))
    o_g = jax.nn.sigmoid(gate(2))
    cc = i_g * t_g
    h_ref[...] = o_g * jnp.tanh(cc)
    c_ref[...] = cc


_gates_call = pl.pallas_call(
    _gates_body,
    grid=(N_PAD // BR,),
    in_specs=[
        pl.BlockSpec((NC, BR, D), lambda i: (0, i, 0)),
        pl.BlockSpec((BR, D), lambda i: (i, 0)),
        pl.BlockSpec((3, D, D), lambda i: (0, 0, 0)),
        pl.BlockSpec((3, D, D), lambda i: (0, 0, 0)),
        pl.BlockSpec((8, D), lambda i: (0, 0)),
        pl.BlockSpec((8, D), lambda i: (0, 0)),
    ],
    out_specs=[
        pl.BlockSpec((BR, D), lambda i: (i, 0)),
        pl.BlockSpec((BR, D), lambda i: (i, 0)),
    ],
    out_shape=[
        jax.ShapeDtypeStruct((N_PAD, D), jnp.float32),
        jax.ShapeDtypeStruct((N_PAD, D), jnp.float32),
    ],
)


def kernel(X, edge_index, edge_weight, Wx, Vx, bx, Wh, Vh, bh, b):
    src = edge_index[0]
    dst = edge_index[1]

    npad = E_PAD - E
    pad_ids = jnp.arange(npad, dtype=jnp.int32)
    src_p = jnp.concatenate([src, pad_ids % N])
    dst_p = jnp.concatenate([dst, N + pad_ids % (N_PAD - N)])
    ew_p = jnp.concatenate([edge_weight,
                            jnp.zeros((npad,), jnp.float32)])

    degp = _deg_call(src_p, ew_p)
    dis = _dis_call(degp.reshape(NC, N_PAD // D, D)).reshape(N_PAD)
    pp = _prop_call(src_p, dst_p, ew_p, dis, X)

    Xp = jnp.pad(X, ((0, N_PAD - N), (0, 0)))
    sel = jnp.array([0, 2, 3])
    bxs = jnp.pad(bx[sel], ((0, 5), (0, 0)))
    addb = jnp.pad(jax.nn.relu(bh[sel]) + b[sel, 0], ((0, 5), (0, 0)))
    H, C = _gates_call(pp, Xp, Wx[sel], Vx[sel], bxs, addb)
    return H[:N], C[:N]


# speedup vs baseline: 65.1080x; 1.8096x over previous
import jax
import jax.numpy as jnp
from jax import lax
from jax.experimental import pallas as pl
from jax.experimental.pallas import tpu as pltpu
from jax.experimental.pallas import tpu_sc as plsc

N = 10000
E = 320000
D = 128

NC = 2
NS = 16
NW = NC * NS
L = 16

CH = 128
RING = 4
MAC = 4
G = 80
E_PAD = NW * G * CH
N_PAD = 10240
RPT = N_PAD // NS

_MESH = dict(core_axis_name="c", subcore_axis_name="s", num_cores=NC,
             num_subcores=NS)

_BCAST_DNUMS = lax.GatherDimensionNumbers(
    offset_dims=(), collapsed_slice_dims=(0,), start_index_map=(0,))


def _bcast(v16, j):
    return lax.gather(v16, jnp.full((L, 1), j, jnp.int32), _BCAST_DNUMS, (1,),
                      mode=lax.GatherScatterMode.PROMISE_IN_BOUNDS)


RING_A = 8


def _deg_body(edges_hbm, degp_hbm, ebuf, ewf, z_v, deg_sh, sem_e, sem_s):
    c = lax.axis_index("c")
    s = lax.axis_index("s")
    wid = s * NC + c
    base = wid * G
    zero16 = jnp.zeros((L,), jnp.float32)

    def zero_z(i, carry):
        z_v[pl.ds(i * L, L)] = zero16
        return carry

    lax.fori_loop(0, RPT // L, zero_z, 0)
    pltpu.sync_copy(z_v, deg_sh.at[pl.ds(s * RPT, RPT)])
    plsc.subcore_barrier()

    def eload(g, b):
        pltpu.async_copy(edges_hbm.at[base + g],
                         ebuf.at[pl.ds(b * 3, 3)], sem_e.at[b])

    def ewait(g, b):
        pltpu.make_async_copy(edges_hbm.at[base + g],
                              ebuf.at[pl.ds(b * 3, 3)], sem_e.at[b]).wait()

    def sissue(b):
        pltpu.async_copy(ewf.at[b], deg_sh.at[ebuf.at[b * 3]],
                         sem_s.at[b], add=True)

    def sdrain(b):
        pltpu.make_async_copy(ewf.at[b], deg_sh.at[ebuf.at[b * 3]],
                              sem_s.at[b]).wait()

    for g in range(6):
        eload(g, g)

    def octet(i, carry):
        g0 = i * RING_A
        for b in range(RING_A):
            g = g0 + b
            ewait(g, b)
            for t in range(CH // L):
                ewf[b, pl.ds(t * L, L)] = plsc.bitcast(
                    ebuf[b * 3 + 2, pl.ds(t * L, L)], jnp.float32)
            sissue(b)

            @pl.when(g >= 2)
            def _():
                sdrain((b + RING_A - 2) % RING_A)

            @pl.when(g + 6 < G)
            def _():
                eload(g + 6, (b + 6) % RING_A)
        return carry

    lax.fori_loop(0, G // RING_A, octet, 0)
    for b in ((G - 2) % RING_A, (G - 1) % RING_A):
        sdrain(b)
    plsc.subcore_barrier()
    pltpu.sync_copy(deg_sh.at[pl.ds(s * RPT, RPT)],
                    degp_hbm.at[c, pl.ds(s * RPT, RPT)])


_deg_call = pl.kernel(
    _deg_body,
    out_type=jax.ShapeDtypeStruct((NC, N_PAD), jnp.float32),
    mesh=plsc.VectorSubcoreMesh(**_MESH),
    scratch_types=[
        pltpu.VMEM((RING_A * 3, CH), jnp.int32),
        pltpu.VMEM((RING_A, CH), jnp.float32),
        pltpu.VMEM((RPT,), jnp.float32),
        pltpu.VMEM_SHARED((N_PAD,), jnp.float32),
        pltpu.SemaphoreType.DMA((RING_A,)),
        pltpu.SemaphoreType.DMA((RING_A,)),
    ],
    compiler_params=pltpu.CompilerParams(needs_layout_passes=False),
)


BR = 1024


def _dis_body(degp3_ref, degpc_ref, x_ref, dis_ref, xs_ref):
    d3 = degp3_ref[0] + degp3_ref[1]
    dis_ref[...] = jnp.where(d3 > 0, lax.rsqrt(d3), 0.0)
    dc = degpc_ref[0] + degpc_ref[1]
    disc = jnp.where(dc > 0, lax.rsqrt(dc), 0.0)
    xs_ref[...] = x_ref[...] * disc


_dis_call = pl.pallas_call(
    _dis_body,
    grid=(N_PAD // BR,),
    in_specs=[
        pl.BlockSpec((NC, BR // D, D), lambda i: (0, i, 0)),
        pl.BlockSpec((NC, BR, 1), lambda i: (0, i, 0)),
        pl.BlockSpec((BR, D), lambda i: (i, 0)),
    ],
    out_specs=[
        pl.BlockSpec((BR // D, D), lambda i: (i, 0)),
        pl.BlockSpec((BR, D), lambda i: (i, 0)),
    ],
    out_shape=[
        jax.ShapeDtypeStruct((N_PAD // D, D), jnp.float32),
        jax.ShapeDtypeStruct((N_PAD, D), jnp.float32),
    ],
)


RING_R = 2


def _prop_body(edges_hbm, xs_hbm, pp_hbm,
               ebuf, rows, acc_sh, sem_e, sem_g, sem_s):
    c = lax.axis_index("c")
    s = lax.axis_index("s")
    wid = s * NC + c
    base = wid * G

    zero16 = jnp.zeros((L,), jnp.float32)

    def zero_rows(r, carry):
        for cs in range(D // L):
            rows[r, pl.ds(cs * L, L)] = zero16
        return carry

    lax.fori_loop(0, CH, zero_rows, 0)
    for t in range(RPT // CH):
        pltpu.sync_copy(rows.at[pl.ds(0, CH)],
                        acc_sh.at[pl.ds(s * RPT + t * CH, CH)])
    plsc.subcore_barrier()

    def eload(g, b):
        pltpu.async_copy(edges_hbm.at[base + g],
                         ebuf.at[pl.ds(b * 3, 3)], sem_e.at[b])

    def ewait(g, b):
        pltpu.make_async_copy(edges_hbm.at[base + g],
                              ebuf.at[pl.ds(b * 3, 3)], sem_e.at[b]).wait()

    def gissue(b, rb):
        pltpu.async_copy(xs_hbm.at[ebuf.at[b * 3]],
                         rows.at[pl.ds(rb * CH, CH)], sem_g.at[rb])

    def gwait(b, rb):
        pltpu.make_async_copy(xs_hbm.at[ebuf.at[b * 3]],
                              rows.at[pl.ds(rb * CH, CH)],
                              sem_g.at[rb]).wait()

    def sissue(b, rb):
        pltpu.async_copy(rows.at[pl.ds(rb * CH, CH)],
                         acc_sh.at[ebuf.at[b * 3 + 1]], sem_s.at[rb],
                         add=True)

    def swait(b, rb):
        pltpu.make_async_copy(rows.at[pl.ds(rb * CH, CH)],
                              acc_sh.at[ebuf.at[b * 3 + 1]],
                              sem_s.at[rb]).wait()

    def scale(b, rb):
        def group(t, carry):
            w16 = plsc.bitcast(ebuf[b * 3 + 2, pl.ds(t * L, L)], jnp.float32)
            for j in range(L):
                wj = _bcast(w16, j)
                for cs in range(D // L):
                    rows[rb * CH + t * L + j, pl.ds(cs * L, L)] = (
                        rows[rb * CH + t * L + j, pl.ds(cs * L, L)] * wj)
            return carry

        lax.fori_loop(0, CH // L, group, 0)

    eload(0, 0)
    eload(1, 1)
    ewait(0, 0)
    gissue(0, 0)

    def quad(i, carry):
        g0 = i * RING
        for b in range(RING):
            g = g0 + b
            rb = b % RING_R
            gwait(b, rb)
            scale(b, rb)
            sissue(b, rb)

            @pl.when(g + 2 < G)
            def _():
                eload(g + 2, (b + 2) % RING)

            @pl.when(g + 1 < G)
            def _():
                nrb = (rb + 1) % RING_R

                @pl.when(g >= 1)
                def _():
                    swait((b + 3) % RING, nrb)

                ewait(g + 1, (b + 1) % RING)
                gissue((b + 1) % RING, nrb)
        return carry

    lax.fori_loop(0, G // RING, quad, 0)
    swait((G - 2) % RING, (G - 2) % RING_R)
    swait((G - 1) % RING, (G - 1) % RING_R)
    plsc.subcore_barrier()
    for t in range(RPT // CH):
        pltpu.sync_copy(acc_sh.at[pl.ds(s * RPT + t * CH, CH)],
                        pp_hbm.at[c, pl.ds(s * RPT + t * CH, CH)])


_prop_call = pl.kernel(
    _prop_body,
    out_type=jax.ShapeDtypeStruct((NC, N_PAD, D), jnp.float32),
    mesh=plsc.VectorSubcoreMesh(**_MESH),
    scratch_types=[
        pltpu.VMEM((RING * 3, CH), jnp.int32),
        pltpu.VMEM((RING_R * CH, D), jnp.float32),
        pltpu.VMEM_SHARED((N_PAD, D), jnp.float32),
        pltpu.SemaphoreType.DMA((RING,)),
        pltpu.SemaphoreType.DMA((RING_R,)),
        pltpu.SemaphoreType.DMA((RING_R,)),
    ],
    compiler_params=pltpu.CompilerParams(needs_layout_passes=False),
)


def _gates_body(pp_ref, dis_ref, x_ref, w_ref, v_ref, bxs_ref, addb_ref,
                h_ref, c_ref):
    P = (pp_ref[0] + pp_ref[1]) * (-dis_ref[...])
    x = x_ref[...]

    def gate(k):
        y = jnp.dot(P, w_ref[k], preferred_element_type=jnp.float32)
        y = y + jnp.dot(x, v_ref[k], preferred_element_type=jnp.float32)
        return jnp.maximum(y + bxs_ref[k][None, :], 0.0) + addb_ref[k][None, :]

    i_g = jax.nn.sigmoid(gate(0))
    t_g = jnp.tanh(gate(1))
    o_g = jax.nn.sigmoid(gate(2))
    cc = i_g * t_g
    h_ref[...] = o_g * jnp.tanh(cc)
    c_ref[...] = cc


_gates_call = pl.pallas_call(
    _gates_body,
    grid=(N_PAD // BR,),
    in_specs=[
        pl.BlockSpec((NC, BR, D), lambda i: (0, i, 0)),
        pl.BlockSpec((BR, 1), lambda i: (i, 0)),
        pl.BlockSpec((BR, D), lambda i: (i, 0)),
        pl.BlockSpec((3, D, D), lambda i: (0, 0, 0)),
        pl.BlockSpec((3, D, D), lambda i: (0, 0, 0)),
        pl.BlockSpec((8, D), lambda i: (0, 0)),
        pl.BlockSpec((8, D), lambda i: (0, 0)),
    ],
    out_specs=[
        pl.BlockSpec((BR, D), lambda i: (i, 0)),
        pl.BlockSpec((BR, D), lambda i: (i, 0)),
    ],
    out_shape=[
        jax.ShapeDtypeStruct((N_PAD, D), jnp.float32),
        jax.ShapeDtypeStruct((N_PAD, D), jnp.float32),
    ],
)


def kernel(X, edge_index, edge_weight, Wx, Vx, bx, Wh, Vh, bh, b):
    src = edge_index[0]
    dst = edge_index[1]

    npad = E_PAD - E
    pad_ids = jnp.arange(npad, dtype=jnp.int32)
    src_p = jnp.concatenate([src, pad_ids % N])
    dst_p = jnp.concatenate([dst, N + pad_ids % (N_PAD - N)])
    ew_bits = lax.bitcast_convert_type(
        jnp.concatenate([edge_weight, jnp.zeros((npad,), jnp.float32)]),
        jnp.int32)
    edges = (jnp.stack([src_p, dst_p, ew_bits], axis=0)
             .reshape(3, NW * G, CH).transpose(1, 0, 2))

    Xp = jnp.pad(X, ((0, N_PAD - N), (0, 0)))
    degp = _deg_call(edges)
    dis2d, Xs = _dis_call(degp.reshape(NC, N_PAD // D, D),
                          degp.reshape(NC, N_PAD, 1), Xp)
    pp = _prop_call(edges, Xs)

    sel = jnp.array([0, 2, 3])
    bxs = jnp.pad(bx[sel], ((0, 5), (0, 0)))
    addb = jnp.pad(jax.nn.relu(bh[sel]) + b[sel, 0], ((0, 5), (0, 0)))
    H, C = _gates_call(pp, dis2d.reshape(N_PAD, 1), Xp, Wx[sel], Vx[sel],
                       bxs, addb)
    return H[:N], C[:N]
